# baseline (device time: 50707 ns/iter reference)
import jax
import jax.numpy as jnp
from jax import lax
from jax.experimental import pallas as pl
from jax.experimental.pallas import tpu as pltpu

N_DEV = 32
N_GRP = 16
B = 2
S = 128
HD = 256
D_OUT = 512


def kernel(x, Wq, K_ext, V_ext, Wo):
    K2 = K_ext.reshape(B, S, HD)
    V2 = V_ext.reshape(B, S, HD)

    def body(x_ref, wq_ref, k_ref, v_ref, wo_ref, out_ref,
             kv_all, q_scr, ctx_scr, send_sems, recv_sems):
        my = lax.axis_index("i")

        barrier_sem = pltpu.get_barrier_semaphore()
        for k in range(1, N_GRP):
            tgt = lax.rem(my + 2 * k, N_DEV)
            pl.semaphore_signal(
                barrier_sem, inc=1,
                device_id=(tgt,), device_id_type=pl.DeviceIdType.MESH,
            )
        pl.semaphore_wait(barrier_sem, N_GRP - 1)

        kv_all[0, 0] = jnp.round(
            jnp.clip(k_ref[...], -4.0, 4.0) * (127.0 / 4.0)
        ).astype(jnp.int8)
        kv_all[0, 1] = jnp.round(
            jnp.clip(v_ref[...], -4.0, 4.0) * (127.0 / 4.0)
        ).astype(jnp.int8)

        xm = x_ref[...].reshape(B * S, D_OUT)
        q = jnp.dot(xm, wq_ref[...], preferred_element_type=jnp.float32)
        q_scr[...] = (q * (0.125 * 4.0 / 127.0)).reshape(B, S, HD).astype(jnp.bfloat16)

        groups = [(b, hh, blk)
                  for b in range(B) for hh in range(4) for blk in range(2)]
        acc = [None] * len(groups)
        den = [None] * len(groups)

        def do_slot(s):
            for g, (b, hh, blk) in enumerate(groups):
                rows = pl.ds(blk * 64, 64)
                cols = pl.ds(hh * 64, 64)
                qt = q_scr[b, rows, cols]
                kt = kv_all[s, 0, b, rows, cols].astype(jnp.bfloat16)
                vt = kv_all[s, 1, b, rows, cols].astype(jnp.bfloat16)
                sc = lax.dot_general(
                    qt, kt, (((1,), (1,)), ((), ())),
                    preferred_element_type=jnp.float32,
                )
                w = jnp.exp(sc)
                part = jnp.dot(w.astype(jnp.bfloat16), vt,
                               preferred_element_type=jnp.float32)
                dsum = jnp.sum(w, axis=-1, keepdims=True)
                if acc[g] is None:
                    acc[g] = part
                    den[g] = dsum
                else:
                    acc[g] = acc[g] + part
                    den[g] = den[g] + dsum

        dist = lambda k: min(2 * k, N_DEV - 2 * k)
        send_order = sorted(range(1, N_GRP), key=dist, reverse=True)
        wait_order = sorted(range(1, N_GRP), key=dist)
        rdmas = {}
        for k in send_order:
            tgt = lax.rem(my + 2 * k, N_DEV)
            r = pltpu.make_async_remote_copy(
                src_ref=kv_all.at[0],
                dst_ref=kv_all.at[k],
                send_sem=send_sems.at[k - 1],
                recv_sem=recv_sems.at[k - 1],
                device_id=(tgt,),
                device_id_type=pl.DeviceIdType.MESH,
            )
            r.start()
            rdmas[k] = r
        do_slot(0)
        for k in wait_order:
            rdmas[k].wait_recv()
            do_slot(k)
        for k in send_order:
            rdmas[k].wait_send()

        for g, (b, hh, blk) in enumerate(groups):
            rows = pl.ds(blk * 64, 64)
            cols = pl.ds(hh * 64, 64)
            ctx_scr[b, rows, cols] = acc[g] * (4.0 / 127.0) / den[g]

        out = jnp.dot(ctx_scr[...].reshape(B * S, HD), wo_ref[...],
                      preferred_element_type=jnp.float32)
        out_ref[...] = out.reshape(B, S, D_OUT)

    return pl.pallas_call(
        body,
        out_shape=jax.ShapeDtypeStruct((B, S, D_OUT), jnp.float32),
        in_specs=[pl.BlockSpec(memory_space=pltpu.VMEM)] * 5,
        out_specs=pl.BlockSpec(memory_space=pltpu.VMEM),
        scratch_shapes=[
            pltpu.VMEM((N_GRP, 2, B, S, HD), jnp.int8),
            pltpu.VMEM((B, S, HD), jnp.bfloat16),
            pltpu.VMEM((B, S, HD), jnp.float32),
            pltpu.SemaphoreType.DMA((N_GRP - 1,)),
            pltpu.SemaphoreType.DMA((N_GRP - 1,)),
        ],
        compiler_params=pltpu.CompilerParams(collective_id=0),
    )(x, Wq, K2, V2, Wo)


# device time: 43119 ns/iter; 1.1760x vs baseline; 1.1760x over previous
import jax
import jax.numpy as jnp
from jax import lax
from jax.experimental import pallas as pl
from jax.experimental.pallas import tpu as pltpu

N_DEV = 32
N_GRP = 16
B = 2
S = 128
HD = 256
D_OUT = 512


def kernel(x, Wq, K_ext, V_ext, Wo):
    K2 = K_ext.reshape(B, S, HD)
    V2 = V_ext.reshape(B, S, HD)

    def body(x_ref, wq_ref, k_ref, v_ref, wo_ref, out_ref,
             k_all, v_all, q_scr, ctx_scr, send_sems, recv_sems):
        my = lax.axis_index("i")

        barrier_sem = pltpu.get_barrier_semaphore()
        for k in range(1, N_GRP):
            tgt = lax.rem(my + 2 * k, N_DEV)
            pl.semaphore_signal(
                barrier_sem, inc=1,
                device_id=(tgt,), device_id_type=pl.DeviceIdType.MESH,
            )
        pl.semaphore_wait(barrier_sem, N_GRP - 1)

        k_all[0] = jnp.round(
            jnp.clip(k_ref[...], -4.0, 4.0) * (127.0 / 4.0)
        ).astype(jnp.int8)
        v_all[0] = jnp.round(
            jnp.clip(v_ref[...], -4.0, 4.0) * (127.0 / 4.0)
        ).astype(jnp.int8)

        xm = x_ref[...].reshape(B * S, D_OUT)
        q = jnp.dot(xm, wq_ref[...], preferred_element_type=jnp.float32)
        q_scr[...] = (q * (0.125 * 4.0 / 127.0)).reshape(B, S, HD).astype(jnp.bfloat16)

        groups = [(b, hh, blk)
                  for b in range(B) for hh in range(4) for blk in range(2)]
        acc = [None] * len(groups)
        den = [None] * len(groups)

        def do_slot(s):
            for g, (b, hh, blk) in enumerate(groups):
                rows = pl.ds(blk * 64, 64)
                cols = pl.ds(hh * 64, 64)
                qt = q_scr[b, rows, cols]
                kt = k_all[s, b, rows, cols].astype(jnp.bfloat16)
                vt = v_all[s, b, rows, cols].astype(jnp.bfloat16)
                sc = lax.dot_general(
                    qt, kt, (((1,), (1,)), ((), ())),
                    preferred_element_type=jnp.float32,
                )
                w = jnp.exp(sc)
                part = jnp.dot(w.astype(jnp.bfloat16), vt,
                               preferred_element_type=jnp.float32)
                dsum = jnp.sum(w, axis=-1, keepdims=True)
                if acc[g] is None:
                    acc[g] = part
                    den[g] = dsum
                else:
                    acc[g] = acc[g] + part
                    den[g] = den[g] + dsum

        flows = [(t, b) for t in range(2) for b in range(B)]
        rdmas = {}
        for k in range(1, N_GRP):
            tgt = lax.rem(my + 2 * k, N_DEV)
            for c, (t, b) in enumerate(flows):
                buf = k_all if t == 0 else v_all
                r = pltpu.make_async_remote_copy(
                    src_ref=buf.at[0, b],
                    dst_ref=buf.at[k, b],
                    send_sem=send_sems.at[c, k - 1],
                    recv_sem=recv_sems.at[c, k - 1],
                    device_id=(tgt,),
                    device_id_type=pl.DeviceIdType.MESH,
                )
                r.start()
                rdmas[(k, c)] = r
        do_slot(0)
        for k in range(1, N_GRP):
            for c in range(len(flows)):
                rdmas[(k, c)].wait_recv()
            do_slot(k)
        for key in rdmas:
            rdmas[key].wait_send()

        for g, (b, hh, blk) in enumerate(groups):
            rows = pl.ds(blk * 64, 64)
            cols = pl.ds(hh * 64, 64)
            ctx_scr[b, rows, cols] = acc[g] * (4.0 / 127.0) / den[g]

        out = jnp.dot(ctx_scr[...].reshape(B * S, HD), wo_ref[...],
                      preferred_element_type=jnp.float32)
        out_ref[...] = out.reshape(B, S, D_OUT)

    return pl.pallas_call(
        body,
        out_shape=jax.ShapeDtypeStruct((B, S, D_OUT), jnp.float32),
        in_specs=[pl.BlockSpec(memory_space=pltpu.VMEM)] * 5,
        out_specs=pl.BlockSpec(memory_space=pltpu.VMEM),
        scratch_shapes=[
            pltpu.VMEM((N_GRP, B, S, HD), jnp.int8),
            pltpu.VMEM((N_GRP, B, S, HD), jnp.int8),
            pltpu.VMEM((B, S, HD), jnp.bfloat16),
            pltpu.VMEM((B, S, HD), jnp.float32),
            pltpu.SemaphoreType.DMA((4, N_GRP - 1)),
            pltpu.SemaphoreType.DMA((4, N_GRP - 1)),
        ],
        compiler_params=pltpu.CompilerParams(collective_id=0),
    )(x, Wq, K2, V2, Wo)


# device time: 42579 ns/iter; 1.1909x vs baseline; 1.0127x over previous
import jax
import jax.numpy as jnp
from jax import lax
from jax.experimental import pallas as pl
from jax.experimental.pallas import tpu as pltpu

N_DEV = 32
N_GRP = 16
B = 2
S = 128
HD = 256
D_OUT = 512


def kernel(x, Wq, K_ext, V_ext, Wo):
    K2 = K_ext.reshape(B, S, HD)
    V2 = V_ext.reshape(B, S, HD)

    def body(x_ref, wq_ref, k_ref, v_ref, wo_ref, out_ref,
             k_all, v_all, q_scr, ctx_scr, send_sems, recv_sems):
        my = lax.axis_index("i")

        barrier_sem = pltpu.get_barrier_semaphore()
        for k in range(1, N_GRP):
            tgt = lax.rem(my + 2 * k, N_DEV)
            pl.semaphore_signal(
                barrier_sem, inc=1,
                device_id=(tgt,), device_id_type=pl.DeviceIdType.MESH,
            )
        pl.semaphore_wait(barrier_sem, N_GRP - 1)

        k_all[0] = jnp.round(
            jnp.clip(k_ref[...], -4.0, 4.0) * (127.0 / 4.0)
        ).astype(jnp.int8)
        v_all[0] = jnp.round(
            jnp.clip(v_ref[...], -4.0, 4.0) * (127.0 / 4.0)
        ).astype(jnp.int8)

        xm = x_ref[...].reshape(B * S, D_OUT)
        q = jnp.dot(xm, wq_ref[...], preferred_element_type=jnp.float32)
        q_scr[...] = (q * (0.125 * 4.0 / 127.0)).reshape(B, S, HD).astype(jnp.bfloat16)

        groups = [(b, hh, blk)
                  for b in range(B) for hh in range(4) for blk in range(2)]
        acc = [None] * len(groups)
        den = [None] * len(groups)

        def do_slot(s):
            for g, (b, hh, blk) in enumerate(groups):
                rows = pl.ds(blk * 64, 64)
                cols = pl.ds(hh * 64, 64)
                qt = q_scr[b, rows, cols]
                kt = k_all[s, b, rows, cols].astype(jnp.bfloat16)
                vt = v_all[s, b, rows, cols].astype(jnp.bfloat16)
                sc = lax.dot_general(
                    qt, kt, (((1,), (1,)), ((), ())),
                    preferred_element_type=jnp.float32,
                )
                w = jnp.exp(sc)
                part = jnp.dot(w.astype(jnp.bfloat16), vt,
                               preferred_element_type=jnp.float32)
                dsum = jnp.sum(w, axis=-1, keepdims=True)
                if acc[g] is None:
                    acc[g] = part
                    den[g] = dsum
                else:
                    acc[g] = acc[g] + part
                    den[g] = den[g] + dsum

        rdmas = []
        for k in range(1, N_GRP):
            tgt = lax.rem(my + 2 * k, N_DEV)
            for c, buf in enumerate((k_all, v_all)):
                r = pltpu.make_async_remote_copy(
                    src_ref=buf.at[0],
                    dst_ref=buf.at[k],
                    send_sem=send_sems.at[c, k - 1],
                    recv_sem=recv_sems.at[c, k - 1],
                    device_id=(tgt,),
                    device_id_type=pl.DeviceIdType.MESH,
                )
                r.start()
                rdmas.append(r)
        do_slot(0)
        for k in range(1, N_GRP):
            rdmas[2 * (k - 1)].wait_recv()
            rdmas[2 * (k - 1) + 1].wait_recv()
            do_slot(k)
        for r in rdmas:
            r.wait_send()

        for g, (b, hh, blk) in enumerate(groups):
            rows = pl.ds(blk * 64, 64)
            cols = pl.ds(hh * 64, 64)
            ctx_scr[b, rows, cols] = acc[g] * (4.0 / 127.0) / den[g]

        out = jnp.dot(ctx_scr[...].reshape(B * S, HD), wo_ref[...],
                      preferred_element_type=jnp.float32)
        out_ref[...] = out.reshape(B, S, D_OUT)

    return pl.pallas_call(
        body,
        out_shape=jax.ShapeDtypeStruct((B, S, D_OUT), jnp.float32),
        in_specs=[pl.BlockSpec(memory_space=pltpu.VMEM)] * 5,
        out_specs=pl.BlockSpec(memory_space=pltpu.VMEM),
        scratch_shapes=[
            pltpu.VMEM((N_GRP, B, S, HD), jnp.int8),
            pltpu.VMEM((N_GRP, B, S, HD), jnp.int8),
            pltpu.VMEM((B, S, HD), jnp.bfloat16),
            pltpu.VMEM((B, S, HD), jnp.float32),
            pltpu.SemaphoreType.DMA((2, N_GRP - 1)),
            pltpu.SemaphoreType.DMA((2, N_GRP - 1)),
        ],
        compiler_params=pltpu.CompilerParams(collective_id=0),
    )(x, Wq, K2, V2, Wo)
